# baseline (speedup 1.0000x reference)
import dataclasses
import functools

import jax
import jax.numpy as jnp
from jax import lax
from jax.experimental import pallas as pl
from jax.experimental.pallas import tpu as pltpu
from jax.experimental.pallas import tpu_sc as plsc

B, K, D_H, NUM_AGENTS = 4096, 100, 128, 1000
A_PAD = 1024
ROWS = 6400
SAMPLES = ROWS // K

NW = 32
LANES = 16
KC = 7
SIGS = 104

SC_COUNT = 2432


def _tc_body(hf_ref, sig_ref, keys_ref, out_ref):
    h = hf_ref[...]
    sig = sig_ref[...]

    agent_iota = lax.broadcasted_iota(jnp.int32, (ROWS, A_PAD), 1)
    oh = (sig == agent_iota).astype(jnp.bfloat16)
    g = jnp.dot(oh, keys_ref[...], preferred_element_type=jnp.float32)

    logits = jnp.sum(g * h, axis=1, keepdims=True)
    e = jnp.exp(logits)

    bi = lax.broadcasted_iota(jnp.int32, (SAMPLES, ROWS), 0)
    mi = lax.broadcasted_iota(jnp.int32, (SAMPLES, ROWS), 1)
    seg = (mi >= bi * K) & (mi < (bi + 1) * K)
    pt = seg.astype(jnp.float32)
    ptb = seg.astype(jnp.bfloat16)

    z = jnp.dot(pt, e, preferred_element_type=jnp.float32)
    eh = (e * h).astype(jnp.bfloat16)
    num = jnp.dot(ptb, eh, preferred_element_type=jnp.float32)
    out_ref[...] = num / z


def _tc_aggregate(h, sigma, keys, b_off, b_count):
    hf = h.reshape(B * K, D_H)
    sigc = sigma.astype(jnp.int32).reshape(B * K, 1)
    keys_pad = jnp.zeros((A_PAD, D_H), jnp.bfloat16).at[:NUM_AGENTS].set(
        keys.astype(jnp.bfloat16))
    off_blocks = (b_off * K) // ROWS

    return pl.pallas_call(
        _tc_body,
        grid=((b_count * K) // ROWS,),
        in_specs=[
            pl.BlockSpec((ROWS, D_H), lambda i: (i + off_blocks, 0)),
            pl.BlockSpec((ROWS, 1), lambda i: (i + off_blocks, 0)),
            pl.BlockSpec((A_PAD, D_H), lambda i: (0, 0)),
        ],
        out_specs=pl.BlockSpec((SAMPLES, D_H), lambda i: (i, 0)),
        out_shape=jax.ShapeDtypeStruct((b_count, D_H), jnp.float32),
    )(hf, sigc, keys_pad)


def _sc_aggregate(h, sigma, keys, b_off, b_count):
    ns_per = b_count // NW
    sigp = jnp.pad(sigma.astype(jnp.int32), ((0, 0), (0, SIGS - K)))
    sigf = sigp.reshape(B * SIGS)
    keys_f = keys.astype(jnp.float32)

    mesh = plsc.VectorSubcoreMesh(core_axis_name="c", subcore_axis_name="s")
    cp = pltpu.CompilerParams()
    if "needs_layout_passes" in pltpu.CompilerParams.__dataclass_fields__:
        cp = dataclasses.replace(cp, needs_layout_passes=False)

    @functools.partial(
        pl.kernel,
        compiler_params=cp,
        out_type=jax.ShapeDtypeStruct((b_count * D_H,), jnp.float32),
        mesh=mesh,
        scratch_types=[
            pltpu.VMEM((ns_per * SIGS,), jnp.int32),
            pltpu.VMEM((K, D_H), jnp.float32),
            pltpu.VMEM((K, D_H), jnp.float32),
            pltpu.VMEM((K, D_H), jnp.float32),
            pltpu.VMEM((K, D_H), jnp.float32),
            pltpu.VMEM((KC * LANES,), jnp.float32),
            pltpu.VMEM((ns_per * D_H,), jnp.float32),
            pltpu.SemaphoreType.DMA,
            pltpu.SemaphoreType.DMA,
            pltpu.SemaphoreType.DMA,
            pltpu.SemaphoreType.DMA,
        ],
    )
    def sc_kernel(h_hbm, sig_hbm, keys_hbm, out_hbm,
                  sig_v, h_v0, h_v1, g_v0, g_v1, lg_v, out_v,
                  sem_h0, sem_h1, sem_g0, sem_g1):
        wid = lax.axis_index("s") * 2 + lax.axis_index("c")
        base = b_off + wid * ns_per

        pltpu.sync_copy(sig_hbm.at[pl.ds(base * SIGS, ns_per * SIGS)], sig_v)

        def issue(j, h_buf, g_buf, sem_h, sem_g):
            jj = jnp.minimum(j, ns_per - 1)
            pltpu.async_copy(h_hbm.at[base + jj], h_buf, sem_h)
            pltpu.async_copy(
                keys_hbm.at[sig_v.at[pl.ds(jj * SIGS, K)]], g_buf, sem_g)

        def drain(h_buf, g_buf, sem_h, sem_g):
            pltpu.make_async_copy(h_hbm.at[0], h_buf, sem_h).wait()
            pltpu.make_async_copy(h_hbm.at[0], g_buf, sem_g).wait()

        lane = lax.iota(jnp.int32, LANES)

        def compute(j, h_v, g_v):
            skew = lane * 8
            for c in range(KC):
                slot = jnp.minimum(lane + c * LANES, K - 1)

                def dot_body(i, acc):
                    for u in range(4):
                        dv = (skew + (i * 4 + u)) & (D_H - 1)
                        hv = plsc.load_gather(h_v, [slot, dv])
                        kv = plsc.load_gather(g_v, [slot, dv])
                        acc = acc + hv * kv
                    return acc

                acc = lax.fori_loop(
                    0, D_H // 4, dot_body, jnp.zeros((LANES,), jnp.float32))
                if c == KC - 1:
                    acc = jnp.where(lane < K - (KC - 1) * LANES, acc,
                                    jnp.float32(-1e9))
                lg_v[pl.ds(c * LANES, LANES)] = acc

            zacc = jnp.zeros((LANES,), jnp.float32)
            for c in range(KC):
                ev = jnp.exp(lg_v[pl.ds(c * LANES, LANES)])
                lg_v[pl.ds(c * LANES, LANES)] = ev
                zacc = zacc + ev
            z = jnp.sum(zacc)
            zinv = 1.0 / (jnp.zeros((LANES,), jnp.float32) + z)
            for c in range(KC):
                lg_v[pl.ds(c * LANES, LANES)] = (
                    lg_v[pl.ds(c * LANES, LANES)] * zinv)

            def pool_body(k, accs):
                a = plsc.load_gather(lg_v, [jnp.zeros((LANES,), jnp.int32) + k])
                return tuple(
                    accs[c8] + a * h_v[k, pl.ds(c8 * LANES, LANES)]
                    for c8 in range(8))

            accs = lax.fori_loop(
                0, K, pool_body,
                tuple(jnp.zeros((LANES,), jnp.float32) for _ in range(8)))
            for c8 in range(8):
                out_v[pl.ds(j * D_H + c8 * LANES, LANES)] = accs[c8]

        issue(0, h_v0, g_v0, sem_h0, sem_g0)

        @pl.loop(0, ns_per, step=2)
        def _(j):
            issue(j + 1, h_v1, g_v1, sem_h1, sem_g1)
            drain(h_v0, g_v0, sem_h0, sem_g0)
            compute(j, h_v0, g_v0)
            issue(j + 2, h_v0, g_v0, sem_h0, sem_g0)
            drain(h_v1, g_v1, sem_h1, sem_g1)
            compute(j + 1, h_v1, g_v1)

        drain(h_v0, g_v0, sem_h0, sem_g0)

        pltpu.sync_copy(
            out_v, out_hbm.at[pl.ds(wid * ns_per * D_H, ns_per * D_H)])

    out = sc_kernel(h, sigf, keys_f)
    return out.reshape(b_count, D_H)


@jax.jit
def kernel(h, sigma, keys):
    if SC_COUNT == 0:
        return _tc_aggregate(h, sigma, keys, 0, B)
    if SC_COUNT == B:
        return _sc_aggregate(h, sigma, keys, 0, B)
    tc_count = B - SC_COUNT
    sc_out = _sc_aggregate(h, sigma, keys, tc_count, SC_COUNT)
    tc_out = _tc_aggregate(h, sigma, keys, 0, tc_count)
    return jnp.concatenate([tc_out, sc_out], axis=0)


# speedup vs baseline: 1.2905x; 1.2905x over previous
import dataclasses
import functools

import jax
import jax.numpy as jnp
from jax import lax
from jax.experimental import pallas as pl
from jax.experimental.pallas import tpu as pltpu
from jax.experimental.pallas import tpu_sc as plsc

B, K, D_H, NUM_AGENTS = 4096, 100, 128, 1000
A_PAD = 1024
ROWS = 6400
SAMPLES = ROWS // K

NW = 32
LANES = 16
KC = 7
SIGS = 104

SC_COUNT = B


def _tc_body(hf_ref, sig_ref, keys_ref, out_ref):
    h = hf_ref[...]
    sig = sig_ref[...]

    agent_iota = lax.broadcasted_iota(jnp.int32, (ROWS, A_PAD), 1)
    oh = (sig == agent_iota).astype(jnp.bfloat16)
    g = jnp.dot(oh, keys_ref[...], preferred_element_type=jnp.float32)

    logits = jnp.sum(g * h, axis=1, keepdims=True)
    e = jnp.exp(logits)

    bi = lax.broadcasted_iota(jnp.int32, (SAMPLES, ROWS), 0)
    mi = lax.broadcasted_iota(jnp.int32, (SAMPLES, ROWS), 1)
    seg = (mi >= bi * K) & (mi < (bi + 1) * K)
    pt = seg.astype(jnp.float32)
    ptb = seg.astype(jnp.bfloat16)

    z = jnp.dot(pt, e, preferred_element_type=jnp.float32)
    eh = (e * h).astype(jnp.bfloat16)
    num = jnp.dot(ptb, eh, preferred_element_type=jnp.float32)
    out_ref[...] = num / z


def _tc_aggregate(h, sigma, keys, b_off, b_count):
    hf = h.reshape(B * K, D_H)
    sigc = sigma.astype(jnp.int32).reshape(B * K, 1)
    keys_pad = jnp.zeros((A_PAD, D_H), jnp.bfloat16).at[:NUM_AGENTS].set(
        keys.astype(jnp.bfloat16))
    off_blocks = (b_off * K) // ROWS

    return pl.pallas_call(
        _tc_body,
        grid=((b_count * K) // ROWS,),
        in_specs=[
            pl.BlockSpec((ROWS, D_H), lambda i: (i + off_blocks, 0)),
            pl.BlockSpec((ROWS, 1), lambda i: (i + off_blocks, 0)),
            pl.BlockSpec((A_PAD, D_H), lambda i: (0, 0)),
        ],
        out_specs=pl.BlockSpec((SAMPLES, D_H), lambda i: (i, 0)),
        out_shape=jax.ShapeDtypeStruct((b_count, D_H), jnp.float32),
    )(hf, sigc, keys_pad)


def _sc_aggregate(h, sigma, keys, b_off, b_count):
    ns_per = b_count // NW
    sigp = jnp.pad(sigma.astype(jnp.int32), ((0, 0), (0, SIGS - K)))
    sigf = sigp.reshape(B * SIGS)
    keys_f = keys.astype(jnp.float32)

    mesh = plsc.VectorSubcoreMesh(core_axis_name="c", subcore_axis_name="s")
    cp = pltpu.CompilerParams()
    if "needs_layout_passes" in pltpu.CompilerParams.__dataclass_fields__:
        cp = dataclasses.replace(cp, needs_layout_passes=False)

    @functools.partial(
        pl.kernel,
        compiler_params=cp,
        out_type=jax.ShapeDtypeStruct((b_count * D_H,), jnp.float32),
        mesh=mesh,
        scratch_types=[
            pltpu.VMEM((ns_per * SIGS,), jnp.int32),
            pltpu.VMEM((K, D_H), jnp.float32),
            pltpu.VMEM((K, D_H), jnp.float32),
            pltpu.VMEM((K, D_H), jnp.float32),
            pltpu.VMEM((K, D_H), jnp.float32),
            pltpu.VMEM((KC * LANES,), jnp.float32),
            pltpu.VMEM((ns_per * D_H,), jnp.float32),
            pltpu.SemaphoreType.DMA,
            pltpu.SemaphoreType.DMA,
            pltpu.SemaphoreType.DMA,
            pltpu.SemaphoreType.DMA,
        ],
    )
    def sc_kernel(h_hbm, sig_hbm, keys_hbm, out_hbm,
                  sig_v, h_v0, h_v1, g_v0, g_v1, lg_v, out_v,
                  sem_h0, sem_h1, sem_g0, sem_g1):
        wid = lax.axis_index("s") * 2 + lax.axis_index("c")
        base = b_off + wid * ns_per

        pltpu.sync_copy(sig_hbm.at[pl.ds(base * SIGS, ns_per * SIGS)], sig_v)

        def issue(j, h_buf, g_buf, sem_h, sem_g):
            jj = jnp.minimum(j, ns_per - 1)
            pltpu.async_copy(h_hbm.at[base + jj], h_buf, sem_h)
            pltpu.async_copy(
                keys_hbm.at[sig_v.at[pl.ds(jj * SIGS, K)]], g_buf, sem_g)

        def drain(h_buf, g_buf, sem_h, sem_g):
            pltpu.make_async_copy(h_hbm.at[0], h_buf, sem_h).wait()
            pltpu.make_async_copy(h_hbm.at[0], g_buf, sem_g).wait()

        lane = lax.iota(jnp.int32, LANES)

        def compute(j, h_v, g_v):
            skew = lane * 8
            for c in range(KC):
                slot = jnp.minimum(lane + c * LANES, K - 1)

                def dot_body(i, acc):
                    base = skew + i * 8
                    for u in range(8):
                        dv = (base + u) & (D_H - 1)
                        hv = plsc.load_gather(h_v, [slot, dv])
                        kv = plsc.load_gather(g_v, [slot, dv])
                        acc = acc + hv * kv
                    return acc

                acc = lax.fori_loop(
                    0, D_H // 8, dot_body, jnp.zeros((LANES,), jnp.float32))
                if c == KC - 1:
                    acc = jnp.where(lane < K - (KC - 1) * LANES, acc,
                                    jnp.float32(-1e9))
                lg_v[pl.ds(c * LANES, LANES)] = acc

            zacc = jnp.zeros((LANES,), jnp.float32)
            for c in range(KC):
                ev = jnp.exp(lg_v[pl.ds(c * LANES, LANES)])
                lg_v[pl.ds(c * LANES, LANES)] = ev
                zacc = zacc + ev
            z = jnp.sum(zacc)
            zinv = 1.0 / (jnp.zeros((LANES,), jnp.float32) + z)
            for c in range(KC):
                lg_v[pl.ds(c * LANES, LANES)] = (
                    lg_v[pl.ds(c * LANES, LANES)] * zinv)

            def pool_body(k2, accs):
                k = k2 * 2
                zero = jnp.zeros((LANES,), jnp.int32)
                a0 = plsc.load_gather(lg_v, [zero + k])
                a1 = plsc.load_gather(lg_v, [zero + (k + 1)])
                return tuple(
                    accs[c8]
                    + a0 * h_v[k, pl.ds(c8 * LANES, LANES)]
                    + a1 * h_v[k + 1, pl.ds(c8 * LANES, LANES)]
                    for c8 in range(8))

            accs = lax.fori_loop(
                0, K // 2, pool_body,
                tuple(jnp.zeros((LANES,), jnp.float32) for _ in range(8)))
            for c8 in range(8):
                out_v[pl.ds(j * D_H + c8 * LANES, LANES)] = accs[c8]

        issue(0, h_v0, g_v0, sem_h0, sem_g0)

        @pl.loop(0, ns_per, step=2)
        def _(j):
            issue(j + 1, h_v1, g_v1, sem_h1, sem_g1)
            drain(h_v0, g_v0, sem_h0, sem_g0)
            compute(j, h_v0, g_v0)
            issue(j + 2, h_v0, g_v0, sem_h0, sem_g0)
            drain(h_v1, g_v1, sem_h1, sem_g1)
            compute(j + 1, h_v1, g_v1)

        drain(h_v0, g_v0, sem_h0, sem_g0)

        pltpu.sync_copy(
            out_v, out_hbm.at[pl.ds(wid * ns_per * D_H, ns_per * D_H)])

    out = sc_kernel(h, sigf, keys_f)
    return out.reshape(b_count, D_H)


@jax.jit
def kernel(h, sigma, keys):
    if SC_COUNT == 0:
        return _tc_aggregate(h, sigma, keys, 0, B)
    if SC_COUNT == B:
        return _sc_aggregate(h, sigma, keys, 0, B)
    tc_count = B - SC_COUNT
    sc_out = _sc_aggregate(h, sigma, keys, tc_count, SC_COUNT)
    tc_out = _tc_aggregate(h, sigma, keys, 0, tc_count)
    return jnp.concatenate([tc_out, sc_out], axis=0)


# speedup vs baseline: 1.2998x; 1.0072x over previous
import dataclasses
import functools

import jax
import jax.numpy as jnp
from jax import lax
from jax.experimental import pallas as pl
from jax.experimental.pallas import tpu as pltpu
from jax.experimental.pallas import tpu_sc as plsc

B, K, D_H, NUM_AGENTS = 4096, 100, 128, 1000
A_PAD = 1024
ROWS = 6400
SAMPLES = ROWS // K

NW = 32
LANES = 16
KC = 7
SIGS = 104

SC_COUNT = B


def _tc_body(hf_ref, sig_ref, keys_ref, out_ref):
    h = hf_ref[...]
    sig = sig_ref[...]

    agent_iota = lax.broadcasted_iota(jnp.int32, (ROWS, A_PAD), 1)
    oh = (sig == agent_iota).astype(jnp.bfloat16)
    g = jnp.dot(oh, keys_ref[...], preferred_element_type=jnp.float32)

    logits = jnp.sum(g * h, axis=1, keepdims=True)
    e = jnp.exp(logits)

    bi = lax.broadcasted_iota(jnp.int32, (SAMPLES, ROWS), 0)
    mi = lax.broadcasted_iota(jnp.int32, (SAMPLES, ROWS), 1)
    seg = (mi >= bi * K) & (mi < (bi + 1) * K)
    pt = seg.astype(jnp.float32)
    ptb = seg.astype(jnp.bfloat16)

    z = jnp.dot(pt, e, preferred_element_type=jnp.float32)
    eh = (e * h).astype(jnp.bfloat16)
    num = jnp.dot(ptb, eh, preferred_element_type=jnp.float32)
    out_ref[...] = num / z


def _tc_aggregate(h, sigma, keys, b_off, b_count):
    hf = h.reshape(B * K, D_H)
    sigc = sigma.astype(jnp.int32).reshape(B * K, 1)
    keys_pad = jnp.zeros((A_PAD, D_H), jnp.bfloat16).at[:NUM_AGENTS].set(
        keys.astype(jnp.bfloat16))
    off_blocks = (b_off * K) // ROWS

    return pl.pallas_call(
        _tc_body,
        grid=((b_count * K) // ROWS,),
        in_specs=[
            pl.BlockSpec((ROWS, D_H), lambda i: (i + off_blocks, 0)),
            pl.BlockSpec((ROWS, 1), lambda i: (i + off_blocks, 0)),
            pl.BlockSpec((A_PAD, D_H), lambda i: (0, 0)),
        ],
        out_specs=pl.BlockSpec((SAMPLES, D_H), lambda i: (i, 0)),
        out_shape=jax.ShapeDtypeStruct((b_count, D_H), jnp.float32),
    )(hf, sigc, keys_pad)


def _sc_aggregate(h, sigma, keys, b_off, b_count):
    ns_per = b_count // NW
    sigp = jnp.pad(sigma.astype(jnp.int32), ((0, 0), (0, SIGS - K)))
    sigf = sigp.reshape(B * SIGS)
    keys_f = keys.astype(jnp.float32)

    mesh = plsc.VectorSubcoreMesh(core_axis_name="c", subcore_axis_name="s")
    cp = pltpu.CompilerParams()
    if "needs_layout_passes" in pltpu.CompilerParams.__dataclass_fields__:
        cp = dataclasses.replace(cp, needs_layout_passes=False)

    @functools.partial(
        pl.kernel,
        compiler_params=cp,
        out_type=jax.ShapeDtypeStruct((b_count * D_H,), jnp.float32),
        mesh=mesh,
        scratch_types=[
            pltpu.VMEM((ns_per * SIGS,), jnp.int32),
            pltpu.VMEM((K, D_H), jnp.float32),
            pltpu.VMEM((K, D_H), jnp.float32),
            pltpu.VMEM((K, D_H), jnp.float32),
            pltpu.VMEM((K, D_H), jnp.float32),
            pltpu.VMEM((KC * LANES,), jnp.float32),
            pltpu.VMEM((ns_per * D_H,), jnp.float32),
            pltpu.SemaphoreType.DMA,
            pltpu.SemaphoreType.DMA,
            pltpu.SemaphoreType.DMA,
            pltpu.SemaphoreType.DMA,
        ],
    )
    def sc_kernel(h_hbm, sig_hbm, keys_hbm, out_hbm,
                  sig_v, h_v0, h_v1, g_v0, g_v1, lg_v, out_v,
                  sem_h0, sem_h1, sem_g0, sem_g1):
        wid = lax.axis_index("s") * 2 + lax.axis_index("c")
        base = b_off + wid * ns_per

        pltpu.sync_copy(sig_hbm.at[pl.ds(base * SIGS, ns_per * SIGS)], sig_v)

        def issue(j, h_buf, g_buf, sem_h, sem_g):
            jj = jnp.minimum(j, ns_per - 1)
            pltpu.async_copy(h_hbm.at[base + jj], h_buf, sem_h)
            pltpu.async_copy(
                keys_hbm.at[sig_v.at[pl.ds(jj * SIGS, K)]], g_buf, sem_g)

        def drain(h_buf, g_buf, sem_h, sem_g):
            pltpu.make_async_copy(h_hbm.at[0], h_buf, sem_h).wait()
            pltpu.make_async_copy(h_hbm.at[0], g_buf, sem_g).wait()

        lane = lax.iota(jnp.int32, LANES)

        def compute(j, h_v, g_v):
            skew = lane * 8
            for c in range(KC):
                slot = jnp.minimum(lane + c * LANES, K - 1)

                def dot_body(i, acc):
                    for u in range(4):
                        dv = (skew + (i * 4 + u)) & (D_H - 1)
                        hv = plsc.load_gather(h_v, [slot, dv])
                        kv = plsc.load_gather(g_v, [slot, dv])
                        acc = acc + hv * kv
                    return acc

                acc = lax.fori_loop(
                    0, D_H // 4, dot_body, jnp.zeros((LANES,), jnp.float32))
                if c == KC - 1:
                    acc = jnp.where(lane < K - (KC - 1) * LANES, acc,
                                    jnp.float32(-1e9))
                lg_v[pl.ds(c * LANES, LANES)] = acc

            zacc = jnp.zeros((LANES,), jnp.float32)
            for c in range(KC):
                ev = jnp.exp(lg_v[pl.ds(c * LANES, LANES)])
                lg_v[pl.ds(c * LANES, LANES)] = ev
                zacc = zacc + ev
            z = jnp.sum(zacc)
            zinv = 1.0 / (jnp.zeros((LANES,), jnp.float32) + z)
            for c in range(KC):
                lg_v[pl.ds(c * LANES, LANES)] = (
                    lg_v[pl.ds(c * LANES, LANES)] * zinv)

            def pool_body(k, accs):
                a = plsc.load_gather(lg_v, [jnp.zeros((LANES,), jnp.int32) + k])
                return tuple(
                    accs[c8] + a * h_v[k, pl.ds(c8 * LANES, LANES)]
                    for c8 in range(8))

            accs = lax.fori_loop(
                0, K, pool_body,
                tuple(jnp.zeros((LANES,), jnp.float32) for _ in range(8)))
            for c8 in range(8):
                out_v[pl.ds(j * D_H + c8 * LANES, LANES)] = accs[c8]

        issue(0, h_v0, g_v0, sem_h0, sem_g0)

        @pl.loop(0, ns_per, step=2)
        def _(j):
            issue(j + 1, h_v1, g_v1, sem_h1, sem_g1)
            drain(h_v0, g_v0, sem_h0, sem_g0)
            compute(j, h_v0, g_v0)
            issue(j + 2, h_v0, g_v0, sem_h0, sem_g0)
            drain(h_v1, g_v1, sem_h1, sem_g1)
            compute(j + 1, h_v1, g_v1)

        drain(h_v0, g_v0, sem_h0, sem_g0)

        pltpu.sync_copy(
            out_v, out_hbm.at[pl.ds(wid * ns_per * D_H, ns_per * D_H)])

    out = sc_kernel(h, sigf, keys_f)
    return out.reshape(b_count, D_H)


@jax.jit
def kernel(h, sigma, keys):
    if SC_COUNT == 0:
        return _tc_aggregate(h, sigma, keys, 0, B)
    if SC_COUNT == B:
        return _sc_aggregate(h, sigma, keys, 0, B)
    tc_count = B - SC_COUNT
    sc_out = _sc_aggregate(h, sigma, keys, tc_count, SC_COUNT)
    tc_out = _tc_aggregate(h, sigma, keys, 0, tc_count)
    return jnp.concatenate([tc_out, sc_out], axis=0)


# speedup vs baseline: 1.3025x; 1.0021x over previous
import dataclasses
import functools

import jax
import jax.numpy as jnp
from jax import lax
from jax.experimental import pallas as pl
from jax.experimental.pallas import tpu as pltpu
from jax.experimental.pallas import tpu_sc as plsc

B, K, D_H, NUM_AGENTS = 4096, 100, 128, 1000
NW = 32
LANES = 16
KC = 7
SIGS = 104


def _sc_aggregate(h, sigma, keys, b_off, b_count):
    ns_per = b_count // NW
    sigp = jnp.pad(sigma.astype(jnp.int32), ((0, 0), (0, SIGS - K)))
    sigf = sigp.reshape(B * SIGS)
    keys_f = keys.astype(jnp.float32)

    mesh = plsc.VectorSubcoreMesh(core_axis_name="c", subcore_axis_name="s")
    cp = pltpu.CompilerParams()
    if "needs_layout_passes" in pltpu.CompilerParams.__dataclass_fields__:
        cp = dataclasses.replace(cp, needs_layout_passes=False)

    @functools.partial(
        pl.kernel,
        compiler_params=cp,
        out_type=jax.ShapeDtypeStruct((b_count * D_H,), jnp.float32),
        mesh=mesh,
        scratch_types=[
            pltpu.VMEM((ns_per * SIGS,), jnp.int32),
            pltpu.VMEM((K, D_H), jnp.float32),
            pltpu.VMEM((K, D_H), jnp.float32),
            pltpu.VMEM((K, D_H), jnp.float32),
            pltpu.VMEM((K, D_H), jnp.float32),
            pltpu.VMEM((KC * LANES,), jnp.float32),
            pltpu.VMEM((ns_per * D_H,), jnp.float32),
            pltpu.SemaphoreType.DMA,
            pltpu.SemaphoreType.DMA,
            pltpu.SemaphoreType.DMA,
            pltpu.SemaphoreType.DMA,
        ],
    )
    def sc_kernel(h_hbm, sig_hbm, keys_hbm, out_hbm,
                  sig_v, h_v0, h_v1, g_v0, g_v1, lg_v, out_v,
                  sem_h0, sem_h1, sem_g0, sem_g1):
        wid = lax.axis_index("s") * 2 + lax.axis_index("c")
        base = b_off + wid * ns_per

        pltpu.sync_copy(sig_hbm.at[pl.ds(base * SIGS, ns_per * SIGS)], sig_v)

        def issue(j, h_buf, g_buf, sem_h, sem_g):
            jj = jnp.minimum(j, ns_per - 1)
            pltpu.async_copy(h_hbm.at[base + jj], h_buf, sem_h)
            pltpu.async_copy(
                keys_hbm.at[sig_v.at[pl.ds(jj * SIGS, K)]], g_buf, sem_g)

        def drain(h_buf, g_buf, sem_h, sem_g):
            pltpu.make_async_copy(h_hbm.at[0], h_buf, sem_h).wait()
            pltpu.make_async_copy(h_hbm.at[0], g_buf, sem_g).wait()

        lane = lax.iota(jnp.int32, LANES)

        def compute(j, h_v, g_v):
            skew = lane * 8
            for c in range(KC):
                slot = jnp.minimum(lane + c * LANES, K - 1)

                def dot_body(i, acc):
                    for u in range(4):
                        dv = (skew + (i * 4 + u)) & (D_H - 1)
                        hv = plsc.load_gather(h_v, [slot, dv])
                        kv = plsc.load_gather(g_v, [slot, dv])
                        acc = acc + hv * kv
                    return acc

                acc = lax.fori_loop(
                    0, D_H // 4, dot_body, jnp.zeros((LANES,), jnp.float32))
                if c == KC - 1:
                    acc = jnp.where(lane < K - (KC - 1) * LANES, acc,
                                    jnp.float32(-1e9))
                lg_v[pl.ds(c * LANES, LANES)] = acc

            zacc = jnp.zeros((LANES,), jnp.float32)
            for c in range(KC):
                ev = jnp.exp(lg_v[pl.ds(c * LANES, LANES)])
                lg_v[pl.ds(c * LANES, LANES)] = ev
                zacc = zacc + ev
            z = jnp.sum(zacc)
            zinv = 1.0 / (jnp.zeros((LANES,), jnp.float32) + z)
            for c in range(KC):
                lg_v[pl.ds(c * LANES, LANES)] = (
                    lg_v[pl.ds(c * LANES, LANES)] * zinv)

            def pool_body(k, accs):
                a = plsc.load_gather(lg_v, [jnp.zeros((LANES,), jnp.int32) + k])
                return tuple(
                    accs[c8] + a * h_v[k, pl.ds(c8 * LANES, LANES)]
                    for c8 in range(8))

            accs = lax.fori_loop(
                0, K, pool_body,
                tuple(jnp.zeros((LANES,), jnp.float32) for _ in range(8)))
            for c8 in range(8):
                out_v[pl.ds(j * D_H + c8 * LANES, LANES)] = accs[c8]

        issue(0, h_v0, g_v0, sem_h0, sem_g0)

        @pl.loop(0, ns_per, step=2)
        def _(j):
            issue(j + 1, h_v1, g_v1, sem_h1, sem_g1)
            drain(h_v0, g_v0, sem_h0, sem_g0)
            compute(j, h_v0, g_v0)
            issue(j + 2, h_v0, g_v0, sem_h0, sem_g0)
            drain(h_v1, g_v1, sem_h1, sem_g1)
            compute(j + 1, h_v1, g_v1)

        drain(h_v0, g_v0, sem_h0, sem_g0)

        pltpu.sync_copy(
            out_v, out_hbm.at[pl.ds(wid * ns_per * D_H, ns_per * D_H)])

    out = sc_kernel(h, sigf, keys_f)
    return out.reshape(b_count, D_H)


@jax.jit
def kernel(h, sigma, keys):
    return _sc_aggregate(h, sigma, keys, 0, B)
